# baseline (device time: 33231 ns/iter reference)
import jax
import jax.numpy as jnp
from jax import lax
from jax.experimental import pallas as pl
from jax.experimental.pallas import tpu as pltpu

N_DEV = 8
N_CHUNKS = 4


def kernel(x, pi):
    _, m, n = x.shape
    rows = m // N_CHUNKS

    def body(pi_ref, x_ref, out_ref, in_buf, send_buf, copy_sems, send_sems, recv_sems):
        my = lax.axis_index("i")
        dst = pi_ref[my]
        src = jnp.int32(0)
        for j in range(N_DEV):
            src = jnp.where(pi_ref[j] == my, jnp.int32(j), src)

        barrier = pltpu.get_barrier_semaphore()
        pl.semaphore_signal(
            barrier, inc=1, device_id=(src,), device_id_type=pl.DeviceIdType.MESH
        )

        copies = []
        for c in range(N_CHUNKS):
            cp = pltpu.make_async_copy(
                x_ref.at[0, pl.ds(c * rows, rows)],
                in_buf.at[c],
                copy_sems.at[c],
            )
            cp.start()
            copies.append(cp)

        pl.semaphore_wait(barrier, 1)

        rdmas = []
        for c in range(N_CHUNKS):
            copies[c].wait()
            send_buf[pl.ds(c * rows, rows)] = in_buf[c].astype(jnp.bfloat16)
            rdma = pltpu.make_async_remote_copy(
                src_ref=send_buf.at[pl.ds(c * rows, rows)],
                dst_ref=out_ref.at[0].at[pl.ds(c * rows, rows)],
                send_sem=send_sems.at[c],
                recv_sem=recv_sems.at[c],
                device_id=(dst,),
                device_id_type=pl.DeviceIdType.MESH,
            )
            rdma.start()
            rdmas.append(rdma)
        for rdma in rdmas:
            rdma.wait()

    return pl.pallas_call(
        body,
        out_shape=jax.ShapeDtypeStruct((1, m, n), jnp.bfloat16),
        in_specs=[
            pl.BlockSpec(memory_space=pltpu.SMEM),
            pl.BlockSpec(memory_space=pltpu.MemorySpace.HBM),
        ],
        out_specs=pl.BlockSpec(memory_space=pltpu.MemorySpace.HBM),
        scratch_shapes=[
            pltpu.VMEM((N_CHUNKS, rows, n), jnp.float32),
            pltpu.VMEM((m, n), jnp.bfloat16),
            pltpu.SemaphoreType.DMA((N_CHUNKS,)),
            pltpu.SemaphoreType.DMA((N_CHUNKS,)),
            pltpu.SemaphoreType.DMA((N_CHUNKS,)),
        ],
        compiler_params=pltpu.CompilerParams(collective_id=0),
    )(pi, x)


# device time: 32483 ns/iter; 1.0230x vs baseline; 1.0230x over previous
import jax
import jax.numpy as jnp
from jax import lax
from jax.experimental import pallas as pl
from jax.experimental.pallas import tpu as pltpu

N_DEV = 8


def kernel(x, pi):
    _, m, n = x.shape
    xb = x.astype(jnp.bfloat16)

    def body(pi_ref, x_ref, out_ref, send_sem, recv_sem):
        my = lax.axis_index("i")
        dst = pi_ref[my]
        src = jnp.int32(0)
        for j in range(N_DEV):
            src = jnp.where(pi_ref[j] == my, jnp.int32(j), src)

        barrier = pltpu.get_barrier_semaphore()
        pl.semaphore_signal(
            barrier, inc=1, device_id=(src,), device_id_type=pl.DeviceIdType.MESH
        )
        pl.semaphore_wait(barrier, 1)

        rdma = pltpu.make_async_remote_copy(
            src_ref=x_ref,
            dst_ref=out_ref,
            send_sem=send_sem,
            recv_sem=recv_sem,
            device_id=(dst,),
            device_id_type=pl.DeviceIdType.MESH,
        )
        rdma.start()
        rdma.wait()

    return pl.pallas_call(
        body,
        out_shape=jax.ShapeDtypeStruct((1, m, n), jnp.bfloat16),
        in_specs=[
            pl.BlockSpec(memory_space=pltpu.SMEM),
            pl.BlockSpec(memory_space=pltpu.VMEM),
        ],
        out_specs=pl.BlockSpec(memory_space=pltpu.VMEM),
        scratch_shapes=[
            pltpu.SemaphoreType.DMA,
            pltpu.SemaphoreType.DMA,
        ],
        compiler_params=pltpu.CompilerParams(collective_id=0),
    )(pi, xb)
